# baseline (speedup 1.0000x reference)
import dataclasses
import functools

import jax
import jax.numpy as jnp
from jax import lax
from jax.experimental import pallas as pl
from jax.experimental.pallas import tpu as pltpu
from jax.experimental.pallas import tpu_sc as plsc

N = 10000
E = 320000
D = 128

NC = 2
NS = 16
NW = NC * NS
B = 80
NB0 = 96
NB1 = 168
E2 = NS * (NB0 + NB1) * B
NPAD = 10240
RPS = NPAD // NS


@functools.cache
def _mesh():
    return plsc.VectorSubcoreMesh(
        core_axis_name="c", subcore_axis_name="s",
        num_cores=NC, num_subcores=NS,
    )


_CP = pltpu.CompilerParams()
if "needs_layout_passes" in pltpu.CompilerParams.__dataclass_fields__:
    _CP = dataclasses.replace(_CP, needs_layout_passes=False)


def _sc_deg(col, ew, zeros128):
    @functools.partial(
        pl.kernel,
        out_type=jax.ShapeDtypeStruct((NC, NPAD, D), jnp.float32),
        mesh=_mesh(),
        scratch_types=[
            [pltpu.VMEM((B,), jnp.int32) for _ in range(3)],
            [pltpu.VMEM((B,), jnp.float32) for _ in range(3)],
            [pltpu.VMEM((B, D), jnp.float32) for _ in range(3)],
            pltpu.VMEM_SHARED((NPAD, D), jnp.float32),
            [pltpu.SemaphoreType.DMA for _ in range(3)],
            [pltpu.SemaphoreType.DMA for _ in range(3)],
            [pltpu.SemaphoreType.DMA for _ in range(3)],
        ],
        compiler_params=_CP,
    )
    def k(col_hbm, ew_hbm, z_hbm, out_hbm, colv, ewv, mb, acc,
          csem, isem, ssem):
        c = lax.axis_index("c")
        s = lax.axis_index("s")
        pltpu.sync_copy(z_hbm, acc.at[pl.ds(s * RPS, RPS)])
        plsc.subcore_barrier()

        def run(nb, gb):
            base = gb * B

            def istart(m, q):
                pltpu.async_copy(ew_hbm.at[pl.ds(base + m * B, B)], ewv[q],
                                 isem[q])

            def iwait(q):
                pltpu.make_async_copy(ew_hbm.at[pl.ds(0, B)], ewv[q],
                                      isem[q]).wait()

            def cstart(m, q):
                pltpu.async_copy(col_hbm.at[pl.ds(base + m * B, B)],
                                 colv[q], csem[q])

            def cwait(q):
                pltpu.make_async_copy(col_hbm.at[pl.ds(0, B)], colv[q],
                                      csem[q]).wait()

            def swait(q):
                pltpu.make_async_copy(z_hbm.at[pl.ds(0, B)], mb[q],
                                      ssem[q]).wait()

            def phase(m, q, q1, q2):
                @pl.when(m + 2 <= nb - 1)
                def _():
                    istart(m + 2, q2)

                @pl.when(m + 1 <= nb - 1)
                def _():
                    @pl.when(m >= 2)
                    def _():
                        swait(q1)

                    cstart(m + 1, q1)

                iwait(q)

                @plsc.parallel_loop(0, B, unroll=4)
                def _(e):
                    w = plsc.load_gather(ewv[q], [lax.broadcast(e, (16,))])
                    for kk in range(D // 16):
                        mb[q][e, pl.ds(kk * 16, 16)] = w

                cwait(q)
                pltpu.async_copy(mb[q], acc.at[colv[q]], ssem[q], add=True)

            cstart(0, 0)
            istart(0, 0)
            istart(1, 1)

            @pl.loop(0, nb // 3)
            def _(j):
                m = 3 * j
                phase(m, 0, 1, 2)
                phase(m + 1, 1, 2, 0)
                phase(m + 2, 2, 0, 1)

            for mm in (nb - 3, nb - 2, nb - 1):
                swait(mm % 3)

        @pl.when(c == 0)
        def _():
            run(NB0, s * NB0)

        @pl.when(c == 1)
        def _():
            run(NB1, NS * NB0 + s * NB1)

        plsc.subcore_barrier()
        pltpu.sync_copy(acc.at[pl.ds(s * RPS, RPS)],
                        out_hbm.at[c, pl.ds(s * RPS, RPS)])

    return k(col, ew, zeros128)


def _sc_pass(g, row, col, ew, zeros128):
    @functools.partial(
        pl.kernel,
        out_type=jax.ShapeDtypeStruct((NC, NPAD, D), jnp.float32),
        mesh=_mesh(),
        scratch_types=[
            [pltpu.VMEM((B,), jnp.int32) for _ in range(3)],
            [pltpu.VMEM((B,), jnp.int32) for _ in range(3)],
            [pltpu.VMEM((B,), jnp.float32) for _ in range(3)],
            [pltpu.VMEM((B, D), jnp.float32) for _ in range(3)],
            pltpu.VMEM_SHARED((NPAD, D), jnp.float32),
            [pltpu.SemaphoreType.DMA for _ in range(3)],
            [pltpu.SemaphoreType.DMA for _ in range(3)],
            [pltpu.SemaphoreType.DMA for _ in range(3)],
            [pltpu.SemaphoreType.DMA for _ in range(3)],
        ],
        compiler_params=_CP,
    )
    def k(g_hbm, row_hbm, col_hbm, ew_hbm, z_hbm, out_hbm,
          colv, rowv, ewv, buf, acc, csem, isem, gsem, ssem):
        c = lax.axis_index("c")
        s = lax.axis_index("s")
        pltpu.sync_copy(z_hbm, acc.at[pl.ds(s * RPS, RPS)])
        plsc.subcore_barrier()

        def run(nb, gb):
            base = gb * B

            def istart(m, q):
                off = base + m * B
                pltpu.async_copy(row_hbm.at[pl.ds(off, B)], rowv[q],
                                 isem[q])
                pltpu.async_copy(ew_hbm.at[pl.ds(off, B)], ewv[q], isem[q])

            def iwait(q):
                pltpu.make_async_copy(row_hbm.at[pl.ds(0, B)], rowv[q],
                                      isem[q]).wait()
                pltpu.make_async_copy(ew_hbm.at[pl.ds(0, B)], ewv[q],
                                      isem[q]).wait()

            def cstart(m, q):
                pltpu.async_copy(col_hbm.at[pl.ds(base + m * B, B)],
                                 colv[q], csem[q])

            def cwait(q):
                pltpu.make_async_copy(col_hbm.at[pl.ds(0, B)], colv[q],
                                      csem[q]).wait()

            def gstart(q):
                pltpu.async_copy(g_hbm.at[rowv[q]], buf[q], gsem[q])

            def gwait(q):
                pltpu.make_async_copy(g_hbm.at[rowv[q]], buf[q],
                                      gsem[q]).wait()

            def swait(q):
                pltpu.make_async_copy(z_hbm.at[pl.ds(0, B)], buf[q],
                                      ssem[q]).wait()

            def phase(m, q, q1, q2):
                @pl.when(m + 2 <= nb - 1)
                def _():
                    istart(m + 2, q2)

                @pl.when(m + 1 <= nb - 1)
                def _():
                    @pl.when(m >= 2)
                    def _():
                        swait(q1)

                    cstart(m + 1, q1)
                    iwait(q1)
                    gstart(q1)

                gwait(q)

                @plsc.parallel_loop(0, B, unroll=4)
                def _(e):
                    w = plsc.load_gather(ewv[q], [lax.broadcast(e, (16,))])
                    for kk in range(D // 16):
                        sl = (e, pl.ds(kk * 16, 16))
                        buf[q][sl] = buf[q][sl] * w

                cwait(q)
                pltpu.async_copy(buf[q], acc.at[colv[q]], ssem[q], add=True)

            cstart(0, 0)
            istart(0, 0)
            istart(1, 1)
            iwait(0)
            gstart(0)

            @pl.loop(0, nb // 3)
            def _(j):
                m = 3 * j
                phase(m, 0, 1, 2)
                phase(m + 1, 1, 2, 0)
                phase(m + 2, 2, 0, 1)

            for mm in (nb - 3, nb - 2, nb - 1):
                swait(mm % 3)

        @pl.when(c == 0)
        def _():
            run(NB0, s * NB0)

        @pl.when(c == 1)
        def _():
            run(NB1, NS * NB0 + s * NB1)

        plsc.subcore_barrier()
        pltpu.sync_copy(acc.at[pl.ds(s * RPS, RPS)],
                        out_hbm.at[c, pl.ds(s * RPS, RPS)])

    return k(g, row, col, ew, zeros128)


BM = 2000


def _tc_matmul(x, W):
    def body(x_ref, w_ref, o_ref):
        o_ref[...] = jnp.dot(x_ref[...], w_ref[...],
                             preferred_element_type=jnp.float32)

    return pl.pallas_call(
        body,
        grid=(N // BM,),
        in_specs=[pl.BlockSpec((BM, D), lambda i: (i, 0)),
                  pl.BlockSpec((D, D), lambda i: (0, 0))],
        out_specs=pl.BlockSpec((BM, D), lambda i: (i, 0)),
        out_shape=jax.ShapeDtypeStruct((N, D), jnp.float32),
    )(x, W)


def _tc_prep(degp, h1):
    def body(d_ref, h_ref, dis_ref, g_ref):
        dis1 = lax.rsqrt(1.0 + d_ref[0, :, 0:1] + d_ref[1, :, 0:1])
        dis_ref[...] = jnp.broadcast_to(dis1, (BM, 16))
        g_ref[...] = h_ref[...] * dis1

    return pl.pallas_call(
        body,
        grid=(N // BM,),
        in_specs=[pl.BlockSpec((NC, BM, D), lambda i: (0, i, 0)),
                  pl.BlockSpec((BM, D), lambda i: (i, 0))],
        out_specs=[pl.BlockSpec((BM, 16), lambda i: (i, 0)),
                   pl.BlockSpec((BM, D), lambda i: (i, 0))],
        out_shape=[jax.ShapeDtypeStruct((N, 16), jnp.float32),
                   jax.ShapeDtypeStruct((N, D), jnp.float32)],
    )(degp, h1)


def _tc_mid(s1, g1, dis, b1, W2):
    def body(s_ref, g_ref, dis_ref, b_ref, w_ref, o_ref):
        dis1 = dis_ref[:, 0:1]
        h = dis1 * (s_ref[0] + s_ref[1] + g_ref[...]) + b_ref[...]
        h = jnp.maximum(h, 0.0)
        o_ref[...] = dis1 * jnp.dot(h, w_ref[...],
                                    preferred_element_type=jnp.float32)

    return pl.pallas_call(
        body,
        grid=(N // BM,),
        in_specs=[pl.BlockSpec((NC, BM, D), lambda i: (0, i, 0)),
                  pl.BlockSpec((BM, D), lambda i: (i, 0)),
                  pl.BlockSpec((BM, 16), lambda i: (i, 0)),
                  pl.BlockSpec((1, D), lambda i: (0, 0)),
                  pl.BlockSpec((D, D), lambda i: (0, 0))],
        out_specs=pl.BlockSpec((BM, D), lambda i: (i, 0)),
        out_shape=jax.ShapeDtypeStruct((N, D), jnp.float32),
    )(s1, g1, dis, b1, W2)


def _tc_final(s2, g2, dis, b2):
    def body(s_ref, g_ref, dis_ref, b_ref, o_ref):
        dis1 = dis_ref[:, 0:1]
        o_ref[...] = dis1 * (s_ref[0] + s_ref[1] + g_ref[...]) + b_ref[...]

    return pl.pallas_call(
        body,
        grid=(N // BM,),
        in_specs=[pl.BlockSpec((NC, BM, D), lambda i: (0, i, 0)),
                  pl.BlockSpec((BM, D), lambda i: (i, 0)),
                  pl.BlockSpec((BM, 16), lambda i: (i, 0)),
                  pl.BlockSpec((1, D), lambda i: (0, 0))],
        out_specs=pl.BlockSpec((BM, D), lambda i: (i, 0)),
        out_shape=jax.ShapeDtypeStruct((N, D), jnp.float32),
    )(s2, g2, dis, b2)


def kernel(x, edge_index, edge_attr, W1, b1, W2, b2):
    pad = E2 - E
    row = jnp.concatenate([edge_index[0], jnp.zeros((pad,), jnp.int32)])
    col = jnp.concatenate([edge_index[1], jnp.zeros((pad,), jnp.int32)])
    ew = jnp.concatenate([edge_attr, jnp.zeros((pad,), jnp.float32)])
    zeros128 = jnp.zeros((RPS, D), jnp.float32)
    b1r = b1.reshape(1, D)
    b2r = b2.reshape(1, D)

    degp = _sc_deg(col, ew, zeros128)
    h1 = _tc_matmul(x, W1)
    dis, g1 = _tc_prep(degp, h1)
    s1 = _sc_pass(g1, row, col, ew, zeros128)
    g2 = _tc_mid(s1, g1, dis, b1r, W2)
    s2 = _sc_pass(g2, row, col, ew, zeros128)
    return _tc_final(s2, g2, dis, b2r)


# speedup vs baseline: 2.9004x; 2.9004x over previous
import dataclasses
import functools

import jax
import jax.numpy as jnp
from jax import lax
from jax.experimental import pallas as pl
from jax.experimental.pallas import tpu as pltpu
from jax.experimental.pallas import tpu_sc as plsc

N = 10000
E = 320000
D = 128

NC = 2
NS = 16
NW = NC * NS
B = 80
NB = 126
EPW = NB * B
E2 = NW * EPW
NPAD = 10240
RPS = NPAD // NS


@functools.cache
def _mesh():
    return plsc.VectorSubcoreMesh(
        core_axis_name="c", subcore_axis_name="s",
        num_cores=NC, num_subcores=NS,
    )


_CP = pltpu.CompilerParams()
if "needs_layout_passes" in pltpu.CompilerParams.__dataclass_fields__:
    _CP = dataclasses.replace(_CP, needs_layout_passes=False)


def _sc_deg(col3, ew, zeros128):
    @functools.partial(
        pl.kernel,
        out_type=jax.ShapeDtypeStruct((NC, NPAD, D), jnp.float32),
        mesh=_mesh(),
        scratch_types=[
            pltpu.VMEM((NB, B), jnp.int32),
            [pltpu.VMEM((B,), jnp.float32) for _ in range(3)],
            [pltpu.VMEM((B, D), jnp.float32) for _ in range(3)],
            pltpu.VMEM_SHARED((NPAD, D), jnp.float32),
            [pltpu.SemaphoreType.DMA for _ in range(3)],
            [pltpu.SemaphoreType.DMA for _ in range(3)],
        ],
        compiler_params=_CP,
    )
    def k(col_hbm, ew_hbm, z_hbm, out_hbm, colv, ewv, mb, acc, isem, ssem):
        c = lax.axis_index("c")
        s = lax.axis_index("s")
        wid = s * NC + c
        base = wid * EPW
        pltpu.sync_copy(col_hbm.at[wid], colv)
        pltpu.sync_copy(z_hbm, acc.at[pl.ds(s * RPS, RPS)])
        plsc.subcore_barrier()

        def istart(m, q):
            pltpu.async_copy(ew_hbm.at[pl.ds(base + m * B, B)], ewv[q],
                             isem[q])

        def iwait(q):
            pltpu.make_async_copy(ew_hbm.at[pl.ds(0, B)], ewv[q],
                                  isem[q]).wait()

        def swait(q):
            pltpu.make_async_copy(z_hbm.at[pl.ds(0, B)], mb[q],
                                  ssem[q]).wait()

        def phase(m, q):
            @pl.when(m + 2 <= NB - 1)
            def _():
                istart(m + 2, (q + 2) % 3)

            @pl.when(m >= 3)
            def _():
                swait(q)

            iwait(q)

            @plsc.parallel_loop(0, B, unroll=4)
            def _(e):
                w = plsc.load_gather(ewv[q], [lax.broadcast(e, (16,))])
                for kk in range(D // 16):
                    mb[q][e, pl.ds(kk * 16, 16)] = w

            pltpu.async_copy(mb[q], acc.at[colv.at[m]], ssem[q], add=True)

        istart(0, 0)
        istart(1, 1)

        @pl.loop(0, NB // 3)
        def _(j):
            m = 3 * j
            phase(m, 0)
            phase(m + 1, 1)
            phase(m + 2, 2)

        for q in range(3):
            swait(q)

        plsc.subcore_barrier()
        pltpu.sync_copy(acc.at[pl.ds(s * RPS, RPS)],
                        out_hbm.at[c, pl.ds(s * RPS, RPS)])

    return k(col3, ew, zeros128)


def _sc_pass(g, row, col3, ew, zeros128):
    @functools.partial(
        pl.kernel,
        out_type=jax.ShapeDtypeStruct((NC, NPAD, D), jnp.float32),
        mesh=_mesh(),
        scratch_types=[
            pltpu.VMEM((NB, B), jnp.int32),
            [pltpu.VMEM((B,), jnp.int32) for _ in range(3)],
            [pltpu.VMEM((B,), jnp.float32) for _ in range(3)],
            [pltpu.VMEM((B, D), jnp.float32) for _ in range(3)],
            pltpu.VMEM_SHARED((NPAD, D), jnp.float32),
            [pltpu.SemaphoreType.DMA for _ in range(3)],
            [pltpu.SemaphoreType.DMA for _ in range(3)],
            [pltpu.SemaphoreType.DMA for _ in range(3)],
        ],
        compiler_params=_CP,
    )
    def k(g_hbm, row_hbm, col_hbm, ew_hbm, z_hbm, out_hbm,
          colv, rowv, ewv, buf, acc, isem, gsem, ssem):
        c = lax.axis_index("c")
        s = lax.axis_index("s")
        wid = s * NC + c
        base = wid * EPW
        pltpu.sync_copy(col_hbm.at[wid], colv)
        pltpu.sync_copy(z_hbm, acc.at[pl.ds(s * RPS, RPS)])
        plsc.subcore_barrier()

        def istart(m, q):
            off = base + m * B
            pltpu.async_copy(row_hbm.at[pl.ds(off, B)], rowv[q], isem[q])
            pltpu.async_copy(ew_hbm.at[pl.ds(off, B)], ewv[q], isem[q])

        def iwait(q):
            pltpu.make_async_copy(row_hbm.at[pl.ds(0, B)], rowv[q],
                                  isem[q]).wait()
            pltpu.make_async_copy(ew_hbm.at[pl.ds(0, B)], ewv[q],
                                  isem[q]).wait()

        def gstart(q):
            pltpu.async_copy(g_hbm.at[rowv[q]], buf[q], gsem[q])

        def gwait(q):
            pltpu.make_async_copy(g_hbm.at[rowv[q]], buf[q], gsem[q]).wait()

        def swait(q):
            pltpu.make_async_copy(z_hbm.at[pl.ds(0, B)], buf[q],
                                  ssem[q]).wait()

        def phase(m, q, q1, q2):
            @pl.when(m + 2 <= NB - 1)
            def _():
                istart(m + 2, q2)

            @pl.when(m + 1 <= NB - 1)
            def _():
                @pl.when(m >= 2)
                def _():
                    swait(q1)

                iwait(q1)
                gstart(q1)

            gwait(q)

            @plsc.parallel_loop(0, B, unroll=4)
            def _(e):
                w = plsc.load_gather(ewv[q], [lax.broadcast(e, (16,))])
                for kk in range(D // 16):
                    sl = (e, pl.ds(kk * 16, 16))
                    buf[q][sl] = buf[q][sl] * w

            pltpu.async_copy(buf[q], acc.at[colv.at[m]], ssem[q], add=True)

        istart(0, 0)
        istart(1, 1)
        iwait(0)
        gstart(0)

        @pl.loop(0, NB // 3)
        def _(j):
            m = 3 * j
            phase(m, 0, 1, 2)
            phase(m + 1, 1, 2, 0)
            phase(m + 2, 2, 0, 1)

        for q in range(3):
            swait(q)

        plsc.subcore_barrier()
        pltpu.sync_copy(acc.at[pl.ds(s * RPS, RPS)],
                        out_hbm.at[c, pl.ds(s * RPS, RPS)])

    return k(g, row, col3, ew, zeros128)


BM = 2000


def _tc_matmul(x, W):
    def body(x_ref, w_ref, o_ref):
        o_ref[...] = jnp.dot(x_ref[...], w_ref[...],
                             preferred_element_type=jnp.float32)

    return pl.pallas_call(
        body,
        grid=(N // BM,),
        in_specs=[pl.BlockSpec((BM, D), lambda i: (i, 0)),
                  pl.BlockSpec((D, D), lambda i: (0, 0))],
        out_specs=pl.BlockSpec((BM, D), lambda i: (i, 0)),
        out_shape=jax.ShapeDtypeStruct((N, D), jnp.float32),
    )(x, W)


def _tc_prep(degp, h1):
    def body(d_ref, h_ref, dis_ref, g_ref):
        dis1 = lax.rsqrt(1.0 + d_ref[0, :, 0:1] + d_ref[1, :, 0:1])
        dis_ref[...] = jnp.broadcast_to(dis1, (BM, 16))
        g_ref[...] = h_ref[...] * dis1

    return pl.pallas_call(
        body,
        grid=(N // BM,),
        in_specs=[pl.BlockSpec((NC, BM, D), lambda i: (0, i, 0)),
                  pl.BlockSpec((BM, D), lambda i: (i, 0))],
        out_specs=[pl.BlockSpec((BM, 16), lambda i: (i, 0)),
                   pl.BlockSpec((BM, D), lambda i: (i, 0))],
        out_shape=[jax.ShapeDtypeStruct((N, 16), jnp.float32),
                   jax.ShapeDtypeStruct((N, D), jnp.float32)],
    )(degp, h1)


def _tc_mid(s1, g1, dis, b1, W2):
    def body(s_ref, g_ref, dis_ref, b_ref, w_ref, o_ref):
        dis1 = dis_ref[:, 0:1]
        h = dis1 * (s_ref[0] + s_ref[1] + g_ref[...]) + b_ref[...]
        h = jnp.maximum(h, 0.0)
        o_ref[...] = dis1 * jnp.dot(h, w_ref[...],
                                    preferred_element_type=jnp.float32)

    return pl.pallas_call(
        body,
        grid=(N // BM,),
        in_specs=[pl.BlockSpec((NC, BM, D), lambda i: (0, i, 0)),
                  pl.BlockSpec((BM, D), lambda i: (i, 0)),
                  pl.BlockSpec((BM, 16), lambda i: (i, 0)),
                  pl.BlockSpec((1, D), lambda i: (0, 0)),
                  pl.BlockSpec((D, D), lambda i: (0, 0))],
        out_specs=pl.BlockSpec((BM, D), lambda i: (i, 0)),
        out_shape=jax.ShapeDtypeStruct((N, D), jnp.float32),
    )(s1, g1, dis, b1, W2)


def _tc_final(s2, g2, dis, b2):
    def body(s_ref, g_ref, dis_ref, b_ref, o_ref):
        dis1 = dis_ref[:, 0:1]
        o_ref[...] = dis1 * (s_ref[0] + s_ref[1] + g_ref[...]) + b_ref[...]

    return pl.pallas_call(
        body,
        grid=(N // BM,),
        in_specs=[pl.BlockSpec((NC, BM, D), lambda i: (0, i, 0)),
                  pl.BlockSpec((BM, D), lambda i: (i, 0)),
                  pl.BlockSpec((BM, 16), lambda i: (i, 0)),
                  pl.BlockSpec((1, D), lambda i: (0, 0))],
        out_specs=pl.BlockSpec((BM, D), lambda i: (i, 0)),
        out_shape=jax.ShapeDtypeStruct((N, D), jnp.float32),
    )(s2, g2, dis, b2)


def kernel(x, edge_index, edge_attr, W1, b1, W2, b2):
    pad = E2 - E
    row = jnp.concatenate([edge_index[0], jnp.zeros((pad,), jnp.int32)])
    col = jnp.concatenate([edge_index[1], jnp.zeros((pad,), jnp.int32)])
    ew = jnp.concatenate([edge_attr, jnp.zeros((pad,), jnp.float32)])
    col3 = col.reshape(NW, NB, B)
    zeros128 = jnp.zeros((RPS, D), jnp.float32)
    b1r = b1.reshape(1, D)
    b2r = b2.reshape(1, D)

    degp = _sc_deg(col3, ew, zeros128)
    h1 = _tc_matmul(x, W1)
    dis, g1 = _tc_prep(degp, h1)
    s1 = _sc_pass(g1, row, col3, ew, zeros128)
    g2 = _tc_mid(s1, g1, dis, b1r, W2)
    s2 = _sc_pass(g2, row, col3, ew, zeros128)
    return _tc_final(s2, g2, dis, b2r)
